# baseline (speedup 1.0000x reference)
import functools

import jax
import jax.numpy as jnp
from jax import lax
from jax.experimental import pallas as pl
from jax.experimental.pallas import tpu as pltpu
from jax.experimental.pallas import tpu_sc as plsc

B, L, D = 4, 4096, 1024
NC, NS = 2, 16
NW = NC * NS
LROWS_W = L // NW
R = 8
XD = 4
NPC = LROWS_W // R
NK = NPC * B


def _sc_add(x_hbm, pos_hbm, out_hbm, xbuf, pbuf, obuf, xs0, xs1, xs2, xs3,
            ps0, ps1, os0, os1, os2, os3):
    xsem = (xs0, xs1, xs2, xs3)
    psem = (ps0, ps1)
    osem = (os0, os1, os2, os3)
    wid = lax.axis_index("s") * NC + lax.axis_index("c")
    lbase = wid * LROWS_W

    def start_x(k, slot):
        bi = k & (B - 1)
        l0 = lbase + (k >> 2) * R
        pltpu.async_copy(x_hbm.at[bi, pl.ds(l0, R), :], xbuf.at[slot],
                         xsem[slot])

    def wait_x(k, slot):
        bi = k & (B - 1)
        l0 = lbase + (k >> 2) * R
        pltpu.make_async_copy(x_hbm.at[bi, pl.ds(l0, R), :], xbuf.at[slot],
                              xsem[slot]).wait()

    def start_pos(pc, slot):
        l0 = lbase + pc * R
        pltpu.async_copy(pos_hbm.at[pl.ds(l0, R), :], pbuf.at[slot],
                         psem[slot])

    def wait_pos(pc, slot):
        l0 = lbase + pc * R
        pltpu.make_async_copy(pos_hbm.at[pl.ds(l0, R), :], pbuf.at[slot],
                              psem[slot]).wait()

    def start_out(k, slot):
        bi = k & (B - 1)
        l0 = lbase + (k >> 2) * R
        pltpu.async_copy(obuf.at[slot], out_hbm.at[bi, pl.ds(l0, R), :],
                         osem[slot])

    def wait_out(k, slot):
        bi = k & (B - 1)
        l0 = lbase + (k >> 2) * R
        pltpu.make_async_copy(obuf.at[slot], out_hbm.at[bi, pl.ds(l0, R), :],
                              osem[slot]).wait()

    start_pos(0, 0)
    for s in range(XD):
        start_x(s, s)

    @pl.loop(0, NPC, step=2)
    def _(pc0):
        for pp in range(2):
            pc = pc0 + pp
            wait_pos(pc, pp)

            @pl.when(pc + 1 < NPC)
            def _():
                start_pos(pc + 1, 1 - pp)

            for b in range(B):
                k = pc * B + b
                slot = b
                wait_x(k, slot)

                @pl.when(k >= XD)
                def _():
                    wait_out(k - XD, slot)

                @plsc.parallel_loop(0, R * D // 16, unroll=16)
                def _(j):
                    r = j >> 6
                    col = (j & (D // 16 - 1)) * 16
                    s = pl.ds(col, 16)
                    obuf[slot, r, s] = xbuf[slot, r, s] + pbuf[pp, r, s]

                start_out(k, slot)

                @pl.when(k + XD < NK)
                def _():
                    start_x(k + XD, slot)

    for s in range(XD):
        wait_out(NK - XD + s, s)


_sc_kernel = functools.partial(
    pl.kernel,
    mesh=plsc.VectorSubcoreMesh(core_axis_name="c", subcore_axis_name="s"),
    out_type=jax.ShapeDtypeStruct((B, L, D), jnp.float32),
    scratch_types=[
        pltpu.VMEM((XD, R, D), jnp.float32),
        pltpu.VMEM((2, R, D), jnp.float32),
        pltpu.VMEM((XD, R, D), jnp.float32),
        pltpu.SemaphoreType.DMA,
        pltpu.SemaphoreType.DMA,
        pltpu.SemaphoreType.DMA,
        pltpu.SemaphoreType.DMA,
        pltpu.SemaphoreType.DMA,
        pltpu.SemaphoreType.DMA,
        pltpu.SemaphoreType.DMA,
        pltpu.SemaphoreType.DMA,
        pltpu.SemaphoreType.DMA,
        pltpu.SemaphoreType.DMA,
    ],
)(_sc_add)


def kernel(x, pos_table):
    return _sc_kernel(x, pos_table)


# speedup vs baseline: 1.0076x; 1.0076x over previous
import functools

import jax
import jax.numpy as jnp
from jax import lax
from jax.experimental import pallas as pl
from jax.experimental.pallas import tpu as pltpu
from jax.experimental.pallas import tpu_sc as plsc

B, L, D = 4, 4096, 1024
NC, NS = 2, 16
NW = NC * NS
LROWS_W = L // NW
R = 8
XD = 8
OD = 4
NPC = LROWS_W // R
NK = NPC * B


def _sc_add(x_hbm, pos_hbm, out_hbm, xbuf, pbuf, obuf, xs0, xs1, xs2, xs3,
            xs4, xs5, xs6, xs7, ps0, ps1, os0, os1, os2, os3):
    xsem = (xs0, xs1, xs2, xs3, xs4, xs5, xs6, xs7)
    psem = (ps0, ps1)
    osem = (os0, os1, os2, os3)
    wid = lax.axis_index("s") * NC + lax.axis_index("c")
    lbase = wid * LROWS_W

    def start_x(k, slot):
        bi = k & (B - 1)
        l0 = lbase + (k >> 2) * R
        pltpu.async_copy(x_hbm.at[bi, pl.ds(l0, R), :], xbuf.at[slot],
                         xsem[slot])

    def wait_x(k, slot):
        bi = k & (B - 1)
        l0 = lbase + (k >> 2) * R
        pltpu.make_async_copy(x_hbm.at[bi, pl.ds(l0, R), :], xbuf.at[slot],
                              xsem[slot]).wait()

    def start_pos(pc, slot):
        l0 = lbase + pc * R
        pltpu.async_copy(pos_hbm.at[pl.ds(l0, R), :], pbuf.at[slot],
                         psem[slot])

    def wait_pos(pc, slot):
        l0 = lbase + pc * R
        pltpu.make_async_copy(pos_hbm.at[pl.ds(l0, R), :], pbuf.at[slot],
                              psem[slot]).wait()

    def start_out(k, slot):
        bi = k & (B - 1)
        l0 = lbase + (k >> 2) * R
        pltpu.async_copy(obuf.at[slot], out_hbm.at[bi, pl.ds(l0, R), :],
                         osem[slot])

    def wait_out(k, slot):
        bi = k & (B - 1)
        l0 = lbase + (k >> 2) * R
        pltpu.make_async_copy(obuf.at[slot], out_hbm.at[bi, pl.ds(l0, R), :],
                              osem[slot]).wait()

    start_pos(0, 0)
    for s in range(XD):
        start_x(s, s)

    @pl.loop(0, NPC, step=2)
    def _(pc0):
        for pp in range(2):
            pc = pc0 + pp
            wait_pos(pc, pp)

            @pl.when(pc + 1 < NPC)
            def _():
                start_pos(pc + 1, 1 - pp)

            for b in range(B):
                k = pc * B + b
                sx = (pp * B + b) & (XD - 1)
                so = b
                wait_x(k, sx)

                @pl.when(k >= OD)
                def _():
                    wait_out(k - OD, so)

                @plsc.parallel_loop(0, R * D // 16, unroll=8)
                def _(j):
                    r = j >> 6
                    col = (j & (D // 16 - 1)) * 16
                    s = pl.ds(col, 16)
                    obuf[so, r, s] = xbuf[sx, r, s] + pbuf[pp, r, s]

                start_out(k, so)

                @pl.when(k + XD < NK)
                def _():
                    start_x(k + XD, sx)

    for s in range(OD):
        wait_out(NK - OD + s, s)


_sc_kernel = functools.partial(
    pl.kernel,
    mesh=plsc.VectorSubcoreMesh(core_axis_name="c", subcore_axis_name="s"),
    out_type=jax.ShapeDtypeStruct((B, L, D), jnp.float32),
    scratch_types=[
        pltpu.VMEM((XD, R, D), jnp.float32),
        pltpu.VMEM((2, R, D), jnp.float32),
        pltpu.VMEM((OD, R, D), jnp.float32),
        pltpu.SemaphoreType.DMA,
        pltpu.SemaphoreType.DMA,
        pltpu.SemaphoreType.DMA,
        pltpu.SemaphoreType.DMA,
        pltpu.SemaphoreType.DMA,
        pltpu.SemaphoreType.DMA,
        pltpu.SemaphoreType.DMA,
        pltpu.SemaphoreType.DMA,
        pltpu.SemaphoreType.DMA,
        pltpu.SemaphoreType.DMA,
        pltpu.SemaphoreType.DMA,
        pltpu.SemaphoreType.DMA,
        pltpu.SemaphoreType.DMA,
        pltpu.SemaphoreType.DMA,
    ],
)(_sc_add)


def kernel(x, pos_table):
    return _sc_kernel(x, pos_table)


# speedup vs baseline: 1.0092x; 1.0016x over previous
import functools

import jax
import jax.numpy as jnp
from jax import lax
from jax.experimental import pallas as pl
from jax.experimental.pallas import tpu as pltpu
from jax.experimental.pallas import tpu_sc as plsc

B, L, D = 4, 4096, 1024
NC, NS = 2, 16
NW = NC * NS
LROWS_W = L // NW
R = 8
XD = 4
OD = 8
NPC = LROWS_W // R
NK = NPC * B


def _sc_add(x_hbm, pos_hbm, out_hbm, xbuf, pbuf, obuf, xs0, xs1, xs2, xs3,
            ps0, ps1, os0, os1, os2, os3, os4, os5, os6, os7):
    xsem = (xs0, xs1, xs2, xs3)
    psem = (ps0, ps1)
    osem = (os0, os1, os2, os3, os4, os5, os6, os7)
    wid = lax.axis_index("s") * NC + lax.axis_index("c")
    lbase = wid * LROWS_W

    def start_x(k, slot):
        bi = k & (B - 1)
        l0 = lbase + (k >> 2) * R
        pltpu.async_copy(x_hbm.at[bi, pl.ds(l0, R), :], xbuf.at[slot],
                         xsem[slot])

    def wait_x(k, slot):
        bi = k & (B - 1)
        l0 = lbase + (k >> 2) * R
        pltpu.make_async_copy(x_hbm.at[bi, pl.ds(l0, R), :], xbuf.at[slot],
                              xsem[slot]).wait()

    def start_pos(pc, slot):
        l0 = lbase + pc * R
        pltpu.async_copy(pos_hbm.at[pl.ds(l0, R), :], pbuf.at[slot],
                         psem[slot])

    def wait_pos(pc, slot):
        l0 = lbase + pc * R
        pltpu.make_async_copy(pos_hbm.at[pl.ds(l0, R), :], pbuf.at[slot],
                              psem[slot]).wait()

    def start_out(k, slot):
        bi = k & (B - 1)
        l0 = lbase + (k >> 2) * R
        pltpu.async_copy(obuf.at[slot], out_hbm.at[bi, pl.ds(l0, R), :],
                         osem[slot])

    def wait_out(k, slot):
        bi = k & (B - 1)
        l0 = lbase + (k >> 2) * R
        pltpu.make_async_copy(obuf.at[slot], out_hbm.at[bi, pl.ds(l0, R), :],
                              osem[slot]).wait()

    start_pos(0, 0)
    for s in range(XD):
        start_x(s, s)

    @pl.loop(0, NPC, step=2)
    def _(pc0):
        for pp in range(2):
            pc = pc0 + pp
            wait_pos(pc, pp)

            @pl.when(pc + 1 < NPC)
            def _():
                start_pos(pc + 1, 1 - pp)

            for b in range(B):
                k = pc * B + b
                slot = b
                so = (pp * B + b) & (OD - 1)
                wait_x(k, slot)

                @pl.when(k >= OD)
                def _():
                    wait_out(k - OD, so)

                @plsc.parallel_loop(0, R * D // 16, unroll=8)
                def _(j):
                    r = j >> 6
                    col = (j & (D // 16 - 1)) * 16
                    s = pl.ds(col, 16)
                    obuf[so, r, s] = xbuf[slot, r, s] + pbuf[pp, r, s]

                start_out(k, so)

                @pl.when(k + XD < NK)
                def _():
                    start_x(k + XD, slot)

    for s in range(OD):
        wait_out(NK - OD + s, s)


_sc_kernel = functools.partial(
    pl.kernel,
    mesh=plsc.VectorSubcoreMesh(core_axis_name="c", subcore_axis_name="s"),
    out_type=jax.ShapeDtypeStruct((B, L, D), jnp.float32),
    scratch_types=[
        pltpu.VMEM((XD, R, D), jnp.float32),
        pltpu.VMEM((2, R, D), jnp.float32),
        pltpu.VMEM((OD, R, D), jnp.float32),
        pltpu.SemaphoreType.DMA,
        pltpu.SemaphoreType.DMA,
        pltpu.SemaphoreType.DMA,
        pltpu.SemaphoreType.DMA,
        pltpu.SemaphoreType.DMA,
        pltpu.SemaphoreType.DMA,
        pltpu.SemaphoreType.DMA,
        pltpu.SemaphoreType.DMA,
        pltpu.SemaphoreType.DMA,
        pltpu.SemaphoreType.DMA,
        pltpu.SemaphoreType.DMA,
        pltpu.SemaphoreType.DMA,
        pltpu.SemaphoreType.DMA,
        pltpu.SemaphoreType.DMA,
    ],
)(_sc_add)


def kernel(x, pos_table):
    return _sc_kernel(x, pos_table)
